# baseline (speedup 1.0000x reference)
import functools

import jax
import jax.numpy as jnp
from jax.experimental import pallas as pl
from jax.experimental.pallas import tpu as pltpu

LAGS = (1, 2, 4, 8, 12)
MAXLAG = 12
PADR = 16
CLAMP_MIN, CLAMP_MAX = -2.5, 2.5


def _body(params_ref, v_ref, e_ref, w1_ref, w2_ref, mm_ref, o_ref,
          a_ref, hist_ref, *, tb, nt, d_f, d_g):
    k = pl.program_id(0)
    p = jax.lax.rem(k, 2)
    q = 1 - p
    d = d_f + d_g
    hlen = hist_ref.shape[1]

    hist = hist_ref[q].astype(jnp.bfloat16)
    mls = jax.lax.dot_general(
        mm_ref[:], hist, (((1,), (0,)), ((), ())),
        preferred_element_type=jnp.float32,
    )
    h = (mls[:tb] + jnp.exp(mls[tb:]) * e_ref[0]) + params_ref[0]
    col = jax.lax.broadcasted_iota(jnp.int32, (1, a_ref.shape[2]), 1)
    m = a_ref[q] * jnp.where(col >= d_f, h, 1.0)
    o_ref[0, 0] = jnp.clip(
        jnp.dot(m, w2_ref[:], preferred_element_type=jnp.float32,
                precision=jax.lax.Precision.DEFAULT),
        CLAMP_MIN, CLAMP_MAX,
    )

    v = v_ref[0]
    r = jnp.dot(v, w1_ref[:], preferred_element_type=jnp.float32,
                precision=jax.lax.Precision.DEFAULT)
    a_ref[p] = jnp.tanh(r)
    s = r[:, d:d + 1]
    tail = jnp.where(
        jax.lax.rem(k, nt) == 0,
        jnp.broadcast_to(s[0:1], (MAXLAG, 1)),
        hist_ref[q, PADR + tb - MAXLAG:PADR + tb],
    )
    hist_ref[p, :PADR] = jnp.concatenate(
        [jnp.zeros((PADR - MAXLAG, 1), jnp.float32), tail], axis=0)
    hist_ref[p, PADR:PADR + tb] = s
    hist_ref[p, PADR + tb:] = jnp.zeros((hlen - PADR - tb, 1), jnp.float32)


@jax.jit
def kernel(visible, W1f, W2f, W1g, W2g, w_mu, b_mu, w_ls, b_ls):
    B, T, N = visible.shape
    d_f = W1f.shape[1]
    d_g = W1g.shape[1]
    d = d_f + d_g
    dp = 48
    tb = 512
    nt = T // tb
    ntot = B * nt
    hlen = ((PADR + tb + 127) // 128) * 128

    eps = jax.random.normal(jax.random.key(42), (1, B, T), jnp.float32)
    eps = eps.reshape(B, T, 1) * jnp.exp(b_ls)
    w1 = jnp.concatenate([
        W1f, W1g, jnp.full((N, 1), 1.0 / N, jnp.float32),
        jnp.zeros((N, dp - d - 1), jnp.float32),
    ], axis=1)
    w2 = jnp.concatenate([
        W2f, W2g, jnp.zeros((dp - d, N), jnp.float32)
    ], axis=0)
    top = sum(w_mu[i] * jnp.eye(tb, hlen, PADR - lag, jnp.float32)
              for i, lag in enumerate(LAGS))
    bot = sum(w_ls[i] * jnp.eye(tb, hlen, PADR - lag, jnp.float32)
              for i, lag in enumerate(LAGS))
    mm = jnp.concatenate([top, bot], axis=0).astype(jnp.bfloat16)
    params = b_mu[None].astype(jnp.float32)

    out = pl.pallas_call(
        functools.partial(_body, tb=tb, nt=nt, d_f=d_f, d_g=d_g),
        grid=(ntot + 1,),
        in_specs=[
            pl.BlockSpec(memory_space=pltpu.SMEM),
            pl.BlockSpec(
                (1, tb, N),
                lambda k: (jnp.minimum(k, ntot - 1) // nt,
                           jnp.minimum(k, ntot - 1) % nt, 0)),
            pl.BlockSpec(
                (1, tb, 1),
                lambda k: (jnp.maximum(k - 1, 0) // nt,
                           jnp.maximum(k - 1, 0) % nt, 0)),
            pl.BlockSpec((N, dp), lambda k: (0, 0)),
            pl.BlockSpec((dp, N), lambda k: (0, 0)),
            pl.BlockSpec((2 * tb, hlen), lambda k: (0, 0)),
        ],
        out_specs=pl.BlockSpec(
            (1, 1, tb, N),
            lambda k: (0, jnp.maximum(k - 1, 0) // nt,
                       jnp.maximum(k - 1, 0) % nt, 0)),
        out_shape=jax.ShapeDtypeStruct((1, B, T, N), jnp.float32),
        scratch_shapes=[
            pltpu.VMEM((2, tb, dp), jnp.float32),
            pltpu.VMEM((2, hlen, 1), jnp.float32),
        ],
        compiler_params=pltpu.CompilerParams(
            dimension_semantics=("arbitrary",),
        ),
    )(params, visible, eps, w1, w2, mm)
    return out


# speedup vs baseline: 2.0523x; 2.0523x over previous
import functools

import jax
import jax.numpy as jnp
from jax.experimental import pallas as pl
from jax.experimental.pallas import tpu as pltpu

LAGS = (1, 2, 4, 8, 12)
MAXLAG = 12
PADR = 16
CLAMP_MIN, CLAMP_MAX = -2.5, 2.5


def _body(params_ref, v_ref, e_ref, w1_ref, w2_ref, mm_ref, o_ref,
          a_ref, hist_ref, *, tb, nt, d_f, d_g):
    k = pl.program_id(0)
    p = jax.lax.rem(k, 2)
    q = 1 - p
    d = d_f + d_g
    hlen = hist_ref.shape[1]

    hist = hist_ref[q].astype(jnp.bfloat16)
    mls = jax.lax.dot_general(
        mm_ref[:], hist, (((1,), (0,)), ((), ())),
        preferred_element_type=jnp.float32,
    )
    e = jnp.transpose(e_ref[0])
    h = (mls[:tb] + jnp.exp(mls[tb:]) * e) + params_ref[0]
    col = jax.lax.broadcasted_iota(jnp.int32, (1, a_ref.shape[2]), 1)
    m = a_ref[q] * jnp.where(col >= d_f, h, 1.0)
    o_ref[0, 0] = jnp.clip(
        jnp.dot(m, w2_ref[:], preferred_element_type=jnp.float32,
                precision=jax.lax.Precision.DEFAULT),
        CLAMP_MIN, CLAMP_MAX,
    )

    v = v_ref[0]
    r = jnp.dot(v, w1_ref[:], preferred_element_type=jnp.float32,
                precision=jax.lax.Precision.DEFAULT)
    a_ref[p] = jnp.tanh(r)
    s = r[:, d:d + 1]
    tail = jnp.where(
        jax.lax.rem(k, nt) == 0,
        jnp.broadcast_to(s[0:1], (MAXLAG, 1)),
        hist_ref[q, PADR + tb - MAXLAG:PADR + tb],
    )
    hist_ref[p, :PADR] = jnp.concatenate(
        [jnp.zeros((PADR - MAXLAG, 1), jnp.float32), tail], axis=0)
    hist_ref[p, PADR:PADR + tb] = s
    hist_ref[p, PADR + tb:] = jnp.zeros((hlen - PADR - tb, 1), jnp.float32)


@jax.jit
def kernel(visible, W1f, W2f, W1g, W2g, w_mu, b_mu, w_ls, b_ls):
    B, T, N = visible.shape
    d_f = W1f.shape[1]
    d_g = W1g.shape[1]
    d = d_f + d_g
    dp = 48
    tb = 512
    nt = T // tb
    ntot = B * nt
    hlen = ((PADR + tb + 127) // 128) * 128

    eps = jax.random.normal(jax.random.key(42), (1, B, T), jnp.float32)
    eps = eps.reshape(B, 1, T) * jnp.exp(b_ls)
    w1 = jnp.concatenate([
        W1f, W1g, jnp.full((N, 1), 1.0 / N, jnp.float32),
        jnp.zeros((N, dp - d - 1), jnp.float32),
    ], axis=1)
    w2 = jnp.concatenate([
        W2f, W2g, jnp.zeros((dp - d, N), jnp.float32)
    ], axis=0)
    top = sum(w_mu[i] * jnp.eye(tb, hlen, PADR - lag, jnp.float32)
              for i, lag in enumerate(LAGS))
    bot = sum(w_ls[i] * jnp.eye(tb, hlen, PADR - lag, jnp.float32)
              for i, lag in enumerate(LAGS))
    mm = jnp.concatenate([top, bot], axis=0).astype(jnp.bfloat16)
    params = b_mu[None].astype(jnp.float32)

    out = pl.pallas_call(
        functools.partial(_body, tb=tb, nt=nt, d_f=d_f, d_g=d_g),
        grid=(ntot + 1,),
        in_specs=[
            pl.BlockSpec(memory_space=pltpu.SMEM),
            pl.BlockSpec(
                (1, tb, N),
                lambda k: (jnp.minimum(k, ntot - 1) // nt,
                           jnp.minimum(k, ntot - 1) % nt, 0)),
            pl.BlockSpec(
                (1, 1, tb),
                lambda k: (jnp.maximum(k - 1, 0) // nt, 0,
                           jnp.maximum(k - 1, 0) % nt)),
            pl.BlockSpec((N, dp), lambda k: (0, 0)),
            pl.BlockSpec((dp, N), lambda k: (0, 0)),
            pl.BlockSpec((2 * tb, hlen), lambda k: (0, 0)),
        ],
        out_specs=pl.BlockSpec(
            (1, 1, tb, N),
            lambda k: (0, jnp.maximum(k - 1, 0) // nt,
                       jnp.maximum(k - 1, 0) % nt, 0)),
        out_shape=jax.ShapeDtypeStruct((1, B, T, N), jnp.float32),
        scratch_shapes=[
            pltpu.VMEM((2, tb, dp), jnp.float32),
            pltpu.VMEM((2, hlen, 1), jnp.float32),
        ],
        compiler_params=pltpu.CompilerParams(
            dimension_semantics=("arbitrary",),
        ),
    )(params, visible, eps, w1, w2, mm)
    return out
